# baseline (speedup 1.0000x reference)
import functools

import jax
import jax.numpy as jnp
import numpy as np
from jax import lax
from jax.experimental import pallas as pl
from jax.experimental.pallas import tpu as pltpu
from jax.experimental.pallas import tpu_sc as plsc

N = 10000
E = 160000
D = 256
DE = 16
TP_OUT = 384
TW = TP_OUT + D
SCAL = 128
NC = 2
NS = 16
NW = NC * NS
LANES = 16

N_PAD = 10240
WIN = N_PAD // NW
SEG = 640
NSEG = E // SEG
CH = 48
ACC_W = WIN * D
PQW = 256
TIW = 384


def _pack_pairs(v):
    r = v.astype(jnp.bfloat16).astype(jnp.float32)
    bits = jax.lax.bitcast_convert_type(r, jnp.uint32)
    k = v.shape[1] // 2
    a = bits[:, :k]
    b = bits[:, k:]
    packed = (jax.lax.shift_right_logical(a, jnp.uint32(16))
              | (b & jnp.uint32(0xFFFF0000)))
    return jax.lax.bitcast_convert_type(packed, jnp.int32)


def _pq_body(x_ref, w_ref, p_ref, q_ref):
    v = jnp.dot(x_ref[...], w_ref[...], preferred_element_type=jnp.float32)
    nb = v.shape[0]
    zpad = jnp.zeros((nb, PQW - TP_OUT // 2), jnp.int32)
    p_ref[...] = jnp.concatenate([_pack_pairs(v[:, :TP_OUT]), zpad], axis=1)
    q_ref[...] = jnp.concatenate([_pack_pairs(v[:, TP_OUT:]), zpad], axis=1)


def _edge_body(ea_ref, we_ref, w1_ref, b1_ref, w2_ref, b2_ref, w3_ref,
               b3_ref, t_ref):
    ea = ea_ref[...]
    rr = jnp.dot(ea, we_ref[...], preferred_element_type=jnp.float32)
    h = jax.nn.silu(jnp.dot(ea, w1_ref[...],
                            preferred_element_type=jnp.float32) + b1_ref[...])
    h = jax.nn.silu(jnp.dot(h, w2_ref[...],
                            preferred_element_type=jnp.float32) + b2_ref[...])
    ww = (jnp.dot(h, w3_ref[...], preferred_element_type=jnp.float32)
          + b3_ref[...])
    nb = ea.shape[0]
    zpad = jnp.zeros((nb, TIW - TW // 2), jnp.int32)
    t_ref[...] = jnp.concatenate(
        [_pack_pairs(rr), _pack_pairs(ww), zpad], axis=1)


def _post_body(a_ref, w_ref, b_ref, o_ref):
    o_ref[...] = jnp.dot(a_ref[...], w_ref[...],
                         preferred_element_type=jnp.float32) + b_ref[...]


def _sc_body(ei_h, p_h, q_h, t_h, acc_h,
             eiv, sel_eid, sel_src, sel_dst, gidx_v,
             buf_p, buf_q, buf_t, acc, sem):
    c = lax.axis_index("c")
    s = lax.axis_index("s")
    w = c * NS + s
    lo = w * WIN

    zvec = jnp.zeros((LANES,), jnp.float32)

    def zero_body(i, _):
        acc[pl.ds(i * LANES, LANES)] = zvec
        return 0

    lax.fori_loop(0, (ACC_W + D) // LANES, zero_body, 0)

    def process_chunks(nch):
        def chunk_body(ci, _):
            for h in range(CH // LANES):
                sv = sel_src[pl.ds(ci * CH + h * LANES, LANES)]
                gidx_v[pl.ds(h * LANES, LANES)] = jnp.minimum(
                    sv, jnp.int32(N_PAD - 1))
            cp_p = pltpu.async_copy(p_h.at[gidx_v], buf_p, sem)
            cp_q = pltpu.async_copy(
                q_h.at[sel_dst.at[pl.ds(ci * CH, CH)]], buf_q, sem)
            cp_t = pltpu.async_copy(
                t_h.at[sel_eid.at[pl.ds(ci * CH, CH)]], buf_t, sem)
            cp_p.wait()
            cp_q.wait()
            cp_t.wait()

            def unpk(ref, j, col):
                v = plsc.bitcast(ref[j, pl.ds(col, LANES)], jnp.bfloat16)
                return plsc.unpack(v, format=plsc.PackFormat.INTERLEAVED,
                                   preferred_element_type=jnp.float32)

            def edge_body(j, _):
                rv = sel_src[pl.ds(ci * CH + j, LANES)]
                base = (rv[0] - lo) * D
                ug = [None] * 8
                ud = [None] * 8
                wg = [None] * 8

                def usum(g):
                    p0, p1 = unpk(buf_p, j, 16 * g)
                    q0, q1 = unpk(buf_q, j, 16 * g)
                    t0, t1 = unpk(buf_t, j, 16 * g)
                    return p0 + q0 + t0, p1 + q1 + t1

                for g in range(8):
                    us, uhi = usum(g)
                    if g < 4:
                        ug[4 + g] = uhi
                    else:
                        ud[g - 4] = uhi
                    ws, wgv = unpk(buf_t, j, TP_OUT // 2 + 16 * g)
                    wg[g] = wgv
                    y = us / (1.0 + jnp.exp(-us)) * ws
                    plsc.addupdate(acc.at[pl.ds(base + 16 * g, LANES)], y)
                for g in range(8, 12):
                    ulo, uhi = usum(g)
                    i1 = g - 8
                    y1 = ud[i1] * wg[i1] / (1.0 + jnp.exp(-ulo))
                    plsc.addupdate(
                        acc.at[pl.ds(base + SCAL + 16 * i1, LANES)], y1)
                    i2 = g - 4
                    y2 = uhi * wg[i2] / (1.0 + jnp.exp(-ug[i2]))
                    plsc.addupdate(
                        acc.at[pl.ds(base + SCAL + 16 * i2, LANES)], y2)
                return 0

            lax.fori_loop(0, CH, edge_body, 0)
            return 0

        lax.fori_loop(0, nch, chunk_body, 0)

    def seg_body(g, cnt):
        ebase = g * SEG
        pltpu.sync_copy(ei_h.at[:, pl.ds(ebase, SEG)], eiv)

        def sel_body(i, cc):
            sv = eiv[0, pl.ds(i * LANES, LANES)]
            m = (sv >= lo) & (sv < lo + WIN)
            inc = m.astype(jnp.int32)
            nhit = jnp.sum(inc)

            @pl.when(nhit > 0)
            def _store():
                dv = eiv[1, pl.ds(i * LANES, LANES)]
                eid = ebase + i * LANES + lax.iota(jnp.int32, LANES)
                pos = cc + plsc.cumsum(inc) - 1
                plsc.store_scatter(sel_eid, [pos], eid, mask=m)
                plsc.store_scatter(sel_src, [pos], sv, mask=m)
                plsc.store_scatter(sel_dst, [pos], dv, mask=m)

            return cc + nhit

        cnt = lax.fori_loop(0, SEG // LANES, sel_body, cnt)

        nfull = cnt // CH
        process_chunks(nfull)

        rem = cnt - nfull * CH

        @pl.when(nfull > 0)
        def _move():
            for h in range(CH // LANES):
                ev = sel_eid[pl.ds(nfull * CH + h * LANES, LANES)]
                sv = sel_src[pl.ds(nfull * CH + h * LANES, LANES)]
                dv = sel_dst[pl.ds(nfull * CH + h * LANES, LANES)]
                sel_eid[pl.ds(h * LANES, LANES)] = ev
                sel_src[pl.ds(h * LANES, LANES)] = sv
                sel_dst[pl.ds(h * LANES, LANES)] = dv

        return rem

    cnt = lax.fori_loop(0, NSEG, seg_body, jnp.int32(0))

    zpad = jnp.zeros((LANES,), jnp.int32)
    tpad = jnp.full((LANES,), lo + WIN, jnp.int32)
    for h in range(CH // LANES):
        sel_eid[pl.ds(cnt + h * LANES, LANES)] = zpad
        sel_src[pl.ds(cnt + h * LANES, LANES)] = tpad
        sel_dst[pl.ds(cnt + h * LANES, LANES)] = zpad
    process_chunks((cnt + CH - 1) // CH)

    pltpu.sync_copy(acc.at[pl.ds(0, ACC_W)], acc_h.at[pl.ds(w * ACC_W, ACC_W)])


def _make_sc_kernel():
    return functools.partial(
        pl.kernel,
        mesh=plsc.VectorSubcoreMesh(core_axis_name="c", subcore_axis_name="s"),
        out_type=jax.ShapeDtypeStruct((N_PAD * D,), jnp.float32),
        scratch_types=[
            pltpu.VMEM((2, SEG), jnp.int32),
            pltpu.VMEM((SEG + 2 * CH,), jnp.int32),
            pltpu.VMEM((SEG + 2 * CH,), jnp.int32),
            pltpu.VMEM((SEG + 2 * CH,), jnp.int32),
            pltpu.VMEM((CH,), jnp.int32),
            pltpu.VMEM((CH, PQW), jnp.int32),
            pltpu.VMEM((CH, PQW), jnp.int32),
            pltpu.VMEM((CH, TIW), jnp.int32),
            pltpu.VMEM((ACC_W + D,), jnp.float32),
            pltpu.SemaphoreType.DMA,
        ],
        compiler_params=pltpu.CompilerParams(needs_layout_passes=False),
    )(_sc_body)


def kernel(x, edge_attr, W_tp, W1, b1, W2, b2, W3, b3, W_post, b_post,
           edge_index):
    x_pad = jnp.pad(x, ((0, N_PAD - N), (0, 0)))
    w_sd = jnp.concatenate([W_tp[:D], W_tp[D:2 * D]], axis=1)
    w_e = W_tp[2 * D:]

    p, q = pl.pallas_call(
        _pq_body,
        grid=(N_PAD // 512,),
        in_specs=[
            pl.BlockSpec((512, D), lambda i: (i, 0)),
            pl.BlockSpec((D, 2 * TP_OUT), lambda i: (0, 0)),
        ],
        out_specs=[
            pl.BlockSpec((512, PQW), lambda i: (i, 0)),
            pl.BlockSpec((512, PQW), lambda i: (i, 0)),
        ],
        out_shape=[
            jax.ShapeDtypeStruct((N_PAD, PQW), jnp.int32),
            jax.ShapeDtypeStruct((N_PAD, PQW), jnp.int32),
        ],
    )(x_pad, w_sd)

    eb = 2000
    t_edges = pl.pallas_call(
        _edge_body,
        grid=(E // eb,),
        in_specs=[
            pl.BlockSpec((eb, DE), lambda i: (i, 0)),
            pl.BlockSpec((DE, TP_OUT), lambda i: (0, 0)),
            pl.BlockSpec((DE, 64), lambda i: (0, 0)),
            pl.BlockSpec((1, 64), lambda i: (0, 0)),
            pl.BlockSpec((64, 64), lambda i: (0, 0)),
            pl.BlockSpec((1, 64), lambda i: (0, 0)),
            pl.BlockSpec((64, D), lambda i: (0, 0)),
            pl.BlockSpec((1, D), lambda i: (0, 0)),
        ],
        out_specs=pl.BlockSpec((eb, TIW), lambda i: (i, 0)),
        out_shape=jax.ShapeDtypeStruct((E, TIW), jnp.int32),
    )(edge_attr, w_e, W1, b1.reshape(1, 64), W2, b2.reshape(1, 64),
      W3, b3.reshape(1, D))

    acc = _make_sc_kernel()(edge_index, p, q, t_edges)
    acc = acc.reshape(N_PAD, D)

    out_pad = pl.pallas_call(
        _post_body,
        grid=(N_PAD // 512,),
        in_specs=[
            pl.BlockSpec((512, D), lambda i: (i, 0)),
            pl.BlockSpec((D, D), lambda i: (0, 0)),
            pl.BlockSpec((1, D), lambda i: (0, 0)),
        ],
        out_specs=pl.BlockSpec((512, D), lambda i: (i, 0)),
        out_shape=jax.ShapeDtypeStruct((N_PAD, D), jnp.float32),
    )(acc, W_post, b_post.reshape(1, D))

    return out_pad[:N]


# speedup vs baseline: 1.1844x; 1.1844x over previous
import functools

import jax
import jax.numpy as jnp
import numpy as np
from jax import lax
from jax.experimental import pallas as pl
from jax.experimental.pallas import tpu as pltpu
from jax.experimental.pallas import tpu_sc as plsc

N = 10000
E = 160000
D = 256
DE = 16
TP_OUT = 384
TW = TP_OUT + D
SCAL = 128
NC = 2
NS = 16
NW = NC * NS
LANES = 16

N_PAD = 10240
WIN = N_PAD // NW
SEG = 640
NSEG = E // SEG
CH = 48
ACC_W = WIN * D
PQW = 256
TIW = 384


def _pack_pairs(v):
    r = v.astype(jnp.bfloat16).astype(jnp.float32)
    bits = jax.lax.bitcast_convert_type(r, jnp.uint32)
    cols = []
    for g in range(v.shape[1] // 32):
        a = bits[:, 32 * g:32 * g + 16]
        b = bits[:, 32 * g + 16:32 * g + 32]
        cols.append(jax.lax.shift_right_logical(a, jnp.uint32(16))
                    | (b & jnp.uint32(0xFFFF0000)))
    packed = jnp.concatenate(cols, axis=1)
    return jax.lax.bitcast_convert_type(packed, jnp.int32)


def _pq_body(x_ref, w_ref, p_ref, q_ref):
    v = jnp.dot(x_ref[...], w_ref[...], preferred_element_type=jnp.float32)
    nb = v.shape[0]
    zpad = jnp.zeros((nb, PQW - TP_OUT // 2), jnp.int32)
    p_ref[...] = jnp.concatenate([_pack_pairs(v[:, :TP_OUT]), zpad], axis=1)
    q_ref[...] = jnp.concatenate([_pack_pairs(v[:, TP_OUT:]), zpad], axis=1)


def _edge_body(ea_ref, we_ref, w1_ref, b1_ref, w2_ref, b2_ref, w3_ref,
               b3_ref, t_ref):
    ea = ea_ref[...]
    rr = jnp.dot(ea, we_ref[...], preferred_element_type=jnp.float32)
    h = jax.nn.silu(jnp.dot(ea, w1_ref[...],
                            preferred_element_type=jnp.float32) + b1_ref[...])
    h = jax.nn.silu(jnp.dot(h, w2_ref[...],
                            preferred_element_type=jnp.float32) + b2_ref[...])
    ww = (jnp.dot(h, w3_ref[...], preferred_element_type=jnp.float32)
          + b3_ref[...])
    nb = ea.shape[0]
    zpad = jnp.zeros((nb, TIW - TW // 2), jnp.int32)
    t_ref[...] = jnp.concatenate(
        [_pack_pairs(rr), _pack_pairs(ww), zpad], axis=1)


def _post_body(a_ref, w_ref, b_ref, o_ref):
    o_ref[...] = jnp.dot(a_ref[...], w_ref[...],
                         preferred_element_type=jnp.float32) + b_ref[...]


def _sc_body(ei_h, p_h, q_h, t_h, acc_h,
             eiv, sel_eid, sel_src, sel_dst, gidx_v,
             buf_p, buf_q, buf_t, acc, sem, sem2):
    c = lax.axis_index("c")
    s = lax.axis_index("s")
    w = c * NS + s
    lo = w * WIN

    pltpu.async_copy(ei_h.at[:, pl.ds(0, SEG)], eiv.at[0], sem2)

    zvec = jnp.zeros((LANES,), jnp.float32)

    def zero_body(i, _):
        acc[pl.ds(i * LANES, LANES)] = zvec
        return 0

    lax.fori_loop(0, (ACC_W + D) // LANES, zero_body, 0)

    def process_chunks(nch):
        def chunk_body(ci, _):
            for h in range(CH // LANES):
                sv = sel_src[pl.ds(ci * CH + h * LANES, LANES)]
                gidx_v[pl.ds(h * LANES, LANES)] = jnp.minimum(
                    sv, jnp.int32(N_PAD - 1))
            cp_p = pltpu.async_copy(p_h.at[gidx_v], buf_p, sem)
            cp_q = pltpu.async_copy(
                q_h.at[sel_dst.at[pl.ds(ci * CH, CH)]], buf_q, sem)
            cp_t = pltpu.async_copy(
                t_h.at[sel_eid.at[pl.ds(ci * CH, CH)]], buf_t, sem)
            cp_p.wait()
            cp_q.wait()
            cp_t.wait()

            def unpk(ref, j, col):
                v = plsc.bitcast(ref[j, pl.ds(col, LANES)], jnp.bfloat16)
                return plsc.unpack(v, format=plsc.PackFormat.INTERLEAVED,
                                   preferred_element_type=jnp.float32)

            def edge_body(j, _):
                rv = sel_src[pl.ds(ci * CH + j, LANES)]
                base = (rv[0] - lo) * D
                for g in range(SCAL // 32):
                    p0, p1 = unpk(buf_p, j, 16 * g)
                    q0, q1 = unpk(buf_q, j, 16 * g)
                    t0, t1 = unpk(buf_t, j, 16 * g)
                    w0, w1 = unpk(buf_t, j, TP_OUT // 2 + 16 * g)
                    u0 = p0 + q0 + t0
                    u1 = p1 + q1 + t1
                    y0 = u0 / (1.0 + jnp.exp(-u0)) * w0
                    y1 = u1 / (1.0 + jnp.exp(-u1)) * w1
                    plsc.addupdate(acc.at[pl.ds(base + 32 * g, LANES)], y0)
                    plsc.addupdate(
                        acc.at[pl.ds(base + 32 * g + LANES, LANES)], y1)
                for g in range(SCAL // 32):
                    pg0, pg1 = unpk(buf_p, j, SCAL // 2 + 16 * g)
                    qg0, qg1 = unpk(buf_q, j, SCAL // 2 + 16 * g)
                    tg0, tg1 = unpk(buf_t, j, SCAL // 2 + 16 * g)
                    pd0, pd1 = unpk(buf_p, j, SCAL + 16 * g)
                    qd0, qd1 = unpk(buf_q, j, SCAL + 16 * g)
                    td0, td1 = unpk(buf_t, j, SCAL + 16 * g)
                    w0, w1 = unpk(buf_t, j, (TP_OUT + SCAL) // 2 + 16 * g)
                    ug0 = pg0 + qg0 + tg0
                    ug1 = pg1 + qg1 + tg1
                    ud0 = pd0 + qd0 + td0
                    ud1 = pd1 + qd1 + td1
                    y0 = ud0 * w0 / (1.0 + jnp.exp(-ug0))
                    y1 = ud1 * w1 / (1.0 + jnp.exp(-ug1))
                    plsc.addupdate(
                        acc.at[pl.ds(base + SCAL + 32 * g, LANES)], y0)
                    plsc.addupdate(
                        acc.at[pl.ds(base + SCAL + 32 * g + LANES, LANES)], y1)
                return 0

            lax.fori_loop(0, CH, edge_body, 0)
            return 0

        lax.fori_loop(0, nch, chunk_body, 0)

    def seg_body(g, cnt):
        ebase = g * SEG
        b = lax.rem(g, 2)
        pltpu.make_async_copy(ei_h.at[:, pl.ds(ebase, SEG)],
                              eiv.at[b], sem2).wait()

        @pl.when(g + 1 < NSEG)
        def _prefetch():
            pltpu.async_copy(ei_h.at[:, pl.ds(ebase + SEG, SEG)],
                             eiv.at[lax.rem(g + 1, 2)], sem2)

        def sel_body(i, cc):
            sv = eiv[b, 0, pl.ds(i * LANES, LANES)]
            m = (sv >= lo) & (sv < lo + WIN)
            inc = m.astype(jnp.int32)
            nhit = jnp.sum(inc)

            @pl.when(nhit > 0)
            def _store():
                dv = eiv[b, 1, pl.ds(i * LANES, LANES)]
                eid = ebase + i * LANES + lax.iota(jnp.int32, LANES)
                pos = cc + plsc.cumsum(inc) - 1
                plsc.store_scatter(sel_eid, [pos], eid, mask=m)
                plsc.store_scatter(sel_src, [pos], sv, mask=m)
                plsc.store_scatter(sel_dst, [pos], dv, mask=m)

            return cc + nhit

        cnt = lax.fori_loop(0, SEG // LANES, sel_body, cnt)

        nfull = cnt // CH
        process_chunks(nfull)

        rem = cnt - nfull * CH

        @pl.when(nfull > 0)
        def _move():
            for h in range(CH // LANES):
                ev = sel_eid[pl.ds(nfull * CH + h * LANES, LANES)]
                sv = sel_src[pl.ds(nfull * CH + h * LANES, LANES)]
                dv = sel_dst[pl.ds(nfull * CH + h * LANES, LANES)]
                sel_eid[pl.ds(h * LANES, LANES)] = ev
                sel_src[pl.ds(h * LANES, LANES)] = sv
                sel_dst[pl.ds(h * LANES, LANES)] = dv

        return rem

    cnt = lax.fori_loop(0, NSEG, seg_body, jnp.int32(0))

    zpad = jnp.zeros((LANES,), jnp.int32)
    tpad = jnp.full((LANES,), lo + WIN, jnp.int32)
    for h in range(CH // LANES):
        sel_eid[pl.ds(cnt + h * LANES, LANES)] = zpad
        sel_src[pl.ds(cnt + h * LANES, LANES)] = tpad
        sel_dst[pl.ds(cnt + h * LANES, LANES)] = zpad
    process_chunks((cnt + CH - 1) // CH)

    pltpu.sync_copy(acc.at[pl.ds(0, ACC_W)], acc_h.at[pl.ds(w * ACC_W, ACC_W)])


def _make_sc_kernel():
    return functools.partial(
        pl.kernel,
        mesh=plsc.VectorSubcoreMesh(core_axis_name="c", subcore_axis_name="s"),
        out_type=jax.ShapeDtypeStruct((N_PAD * D,), jnp.float32),
        scratch_types=[
            pltpu.VMEM((2, 2, SEG), jnp.int32),
            pltpu.VMEM((SEG + 2 * CH,), jnp.int32),
            pltpu.VMEM((SEG + 2 * CH,), jnp.int32),
            pltpu.VMEM((SEG + 2 * CH,), jnp.int32),
            pltpu.VMEM((CH,), jnp.int32),
            pltpu.VMEM((CH, PQW), jnp.int32),
            pltpu.VMEM((CH, PQW), jnp.int32),
            pltpu.VMEM((CH, TIW), jnp.int32),
            pltpu.VMEM((ACC_W + D,), jnp.float32),
            pltpu.SemaphoreType.DMA,
            pltpu.SemaphoreType.DMA,
        ],
        compiler_params=pltpu.CompilerParams(needs_layout_passes=False),
    )(_sc_body)


def kernel(x, edge_attr, W_tp, W1, b1, W2, b2, W3, b3, W_post, b_post,
           edge_index):
    x_pad = jnp.pad(x, ((0, N_PAD - N), (0, 0)))
    w_sd = jnp.concatenate([W_tp[:D], W_tp[D:2 * D]], axis=1)
    w_e = W_tp[2 * D:]

    p, q = pl.pallas_call(
        _pq_body,
        grid=(N_PAD // 512,),
        in_specs=[
            pl.BlockSpec((512, D), lambda i: (i, 0)),
            pl.BlockSpec((D, 2 * TP_OUT), lambda i: (0, 0)),
        ],
        out_specs=[
            pl.BlockSpec((512, PQW), lambda i: (i, 0)),
            pl.BlockSpec((512, PQW), lambda i: (i, 0)),
        ],
        out_shape=[
            jax.ShapeDtypeStruct((N_PAD, PQW), jnp.int32),
            jax.ShapeDtypeStruct((N_PAD, PQW), jnp.int32),
        ],
    )(x_pad, w_sd)

    eb = 2000
    t_edges = pl.pallas_call(
        _edge_body,
        grid=(E // eb,),
        in_specs=[
            pl.BlockSpec((eb, DE), lambda i: (i, 0)),
            pl.BlockSpec((DE, TP_OUT), lambda i: (0, 0)),
            pl.BlockSpec((DE, 64), lambda i: (0, 0)),
            pl.BlockSpec((1, 64), lambda i: (0, 0)),
            pl.BlockSpec((64, 64), lambda i: (0, 0)),
            pl.BlockSpec((1, 64), lambda i: (0, 0)),
            pl.BlockSpec((64, D), lambda i: (0, 0)),
            pl.BlockSpec((1, D), lambda i: (0, 0)),
        ],
        out_specs=pl.BlockSpec((eb, TIW), lambda i: (i, 0)),
        out_shape=jax.ShapeDtypeStruct((E, TIW), jnp.int32),
    )(edge_attr, w_e, W1, b1.reshape(1, 64), W2, b2.reshape(1, 64),
      W3, b3.reshape(1, D))

    acc = _make_sc_kernel()(edge_index, p, q, t_edges)
    acc = acc.reshape(N_PAD, D)

    out_pad = pl.pallas_call(
        _post_body,
        grid=(N_PAD // 512,),
        in_specs=[
            pl.BlockSpec((512, D), lambda i: (i, 0)),
            pl.BlockSpec((D, D), lambda i: (0, 0)),
            pl.BlockSpec((1, D), lambda i: (0, 0)),
        ],
        out_specs=pl.BlockSpec((512, D), lambda i: (i, 0)),
        out_shape=jax.ShapeDtypeStruct((N_PAD, D), jnp.float32),
    )(acc, W_post, b_post.reshape(1, D))

    return out_pad[:N]


# speedup vs baseline: 1.4065x; 1.1875x over previous
import functools

import jax
import jax.numpy as jnp
import numpy as np
from jax import lax
from jax.experimental import pallas as pl
from jax.experimental.pallas import tpu as pltpu
from jax.experimental.pallas import tpu_sc as plsc

N = 10000
E = 160000
D = 256
DE = 16
TP_OUT = 384
TW = TP_OUT + D
SCAL = 128
NC = 2
NS = 16
NW = NC * NS
LANES = 16

N_PAD = 10240
WIN = N_PAD // NW
SEG = 640
NSEG = E // SEG
CH = 48
ACC_W = WIN * D
PQW = 256
TIW = 384


def _pack_pairs(v):
    r = v.astype(jnp.bfloat16).astype(jnp.float32)
    bits = jax.lax.bitcast_convert_type(r, jnp.uint32)
    cols = []
    for rg in range(v.shape[1] // 128):
        a = bits[:, 128 * rg:128 * rg + 64]
        b = bits[:, 128 * rg + 64:128 * rg + 128]
        cols.append(jax.lax.shift_right_logical(a, jnp.uint32(16))
                    | (b & jnp.uint32(0xFFFF0000)))
    packed = jnp.concatenate(cols, axis=1)
    return jax.lax.bitcast_convert_type(packed, jnp.int32)


def _pq_body(x_ref, w_ref, p_ref, q_ref):
    v = jnp.dot(x_ref[...], w_ref[...], preferred_element_type=jnp.float32)
    nb = v.shape[0]
    zpad = jnp.zeros((nb, PQW - TP_OUT // 2), jnp.int32)
    p_ref[...] = jnp.concatenate([_pack_pairs(v[:, :TP_OUT]), zpad], axis=1)
    q_ref[...] = jnp.concatenate([_pack_pairs(v[:, TP_OUT:]), zpad], axis=1)


def _edge_body(ea_ref, we_ref, w1_ref, b1_ref, w2_ref, b2_ref, w3_ref,
               b3_ref, t_ref):
    ea = ea_ref[...]
    rr = jnp.dot(ea, we_ref[...], preferred_element_type=jnp.float32)
    h = jax.nn.silu(jnp.dot(ea, w1_ref[...],
                            preferred_element_type=jnp.float32) + b1_ref[...])
    h = jax.nn.silu(jnp.dot(h, w2_ref[...],
                            preferred_element_type=jnp.float32) + b2_ref[...])
    ww = (jnp.dot(h, w3_ref[...], preferred_element_type=jnp.float32)
          + b3_ref[...])
    nb = ea.shape[0]
    zpad = jnp.zeros((nb, TIW - TW // 2), jnp.int32)
    t_ref[...] = jnp.concatenate(
        [_pack_pairs(rr), _pack_pairs(ww), zpad], axis=1)


def _post_body(a_ref, w_ref, b_ref, o_ref):
    o_ref[...] = jnp.dot(a_ref[...], w_ref[...],
                         preferred_element_type=jnp.float32) + b_ref[...]


def _sc_body(ei_h, p_h, q_h, t_h, acc_h,
             eiv, sel_eid, sel_src, sel_dst, gidx_v,
             buf_p, buf_q, buf_t, acc, sem, sem2):
    c = lax.axis_index("c")
    s = lax.axis_index("s")
    w = c * NS + s
    lo = w * WIN

    pltpu.async_copy(ei_h.at[:, pl.ds(0, SEG)], eiv.at[0], sem2)

    zvec = jnp.zeros((LANES,), jnp.float32)

    def zero_body(i, _):
        acc[pl.ds(i * LANES, LANES)] = zvec
        return 0

    lax.fori_loop(0, (ACC_W + D) // LANES, zero_body, 0)

    def process_chunks(nch):
        def chunk_body(ci, _):
            for h in range(CH // LANES):
                sv = sel_src[pl.ds(ci * CH + h * LANES, LANES)]
                gidx_v[pl.ds(h * LANES, LANES)] = jnp.minimum(
                    sv, jnp.int32(N_PAD - 1))
            cp_p = pltpu.async_copy(p_h.at[gidx_v], buf_p, sem)
            cp_q = pltpu.async_copy(
                q_h.at[sel_dst.at[pl.ds(ci * CH, CH)]], buf_q, sem)
            cp_t = pltpu.async_copy(
                t_h.at[sel_eid.at[pl.ds(ci * CH, CH)]], buf_t, sem)
            cp_p.wait()
            cp_q.wait()
            cp_t.wait()

            def unpk(ref, j, col):
                v = plsc.bitcast(ref[j, pl.ds(col, LANES)], jnp.bfloat16)
                return plsc.unpack(v, format=plsc.PackFormat.INTERLEAVED,
                                   preferred_element_type=jnp.float32)

            def edge_body(j, _):
                rv = sel_src[pl.ds(ci * CH + j, LANES)]
                base = (rv[0] - lo) * D
                for g in range(4):
                    p0, p1 = unpk(buf_p, j, 16 * g)
                    q0, q1 = unpk(buf_q, j, 16 * g)
                    t0, t1 = unpk(buf_t, j, 16 * g)
                    w0, w1 = unpk(buf_t, j, TP_OUT // 2 + 16 * g)
                    u0 = p0 + q0 + t0
                    u1 = p1 + q1 + t1
                    y0 = u0 / (1.0 + jnp.exp(-u0)) * w0
                    y1 = u1 / (1.0 + jnp.exp(-u1)) * w1
                    plsc.addupdate(acc.at[pl.ds(base + 16 * g, LANES)], y0)
                    plsc.addupdate(
                        acc.at[pl.ds(base + 16 * g + 64, LANES)], y1)
                for g in range(4):
                    pg0, pg1 = unpk(buf_p, j, SCAL // 2 + 16 * g)
                    qg0, qg1 = unpk(buf_q, j, SCAL // 2 + 16 * g)
                    tg0, tg1 = unpk(buf_t, j, SCAL // 2 + 16 * g)
                    pd0, pd1 = unpk(buf_p, j, SCAL + 16 * g)
                    qd0, qd1 = unpk(buf_q, j, SCAL + 16 * g)
                    td0, td1 = unpk(buf_t, j, SCAL + 16 * g)
                    w0, w1 = unpk(buf_t, j, (TP_OUT + SCAL) // 2 + 16 * g)
                    ug0 = pg0 + qg0 + tg0
                    ug1 = pg1 + qg1 + tg1
                    ud0 = pd0 + qd0 + td0
                    ud1 = pd1 + qd1 + td1
                    y0 = ud0 * w0 / (1.0 + jnp.exp(-ug0))
                    y1 = ud1 * w1 / (1.0 + jnp.exp(-ug1))
                    plsc.addupdate(
                        acc.at[pl.ds(base + SCAL + 16 * g, LANES)], y0)
                    plsc.addupdate(
                        acc.at[pl.ds(base + SCAL + 16 * g + 64, LANES)], y1)
                return 0

            lax.fori_loop(0, CH, edge_body, 0)
            return 0

        lax.fori_loop(0, nch, chunk_body, 0)

    def seg_body(g, cnt):
        ebase = g * SEG
        b = lax.rem(g, 2)
        pltpu.make_async_copy(ei_h.at[:, pl.ds(ebase, SEG)],
                              eiv.at[b], sem2).wait()

        @pl.when(g + 1 < NSEG)
        def _prefetch():
            pltpu.async_copy(ei_h.at[:, pl.ds(ebase + SEG, SEG)],
                             eiv.at[lax.rem(g + 1, 2)], sem2)

        def sel_body(i, cc):
            sv = eiv[b, 0, pl.ds(i * LANES, LANES)]
            m = (sv >= lo) & (sv < lo + WIN)
            inc = m.astype(jnp.int32)
            nhit = jnp.sum(inc)

            @pl.when(nhit > 0)
            def _store():
                dv = eiv[b, 1, pl.ds(i * LANES, LANES)]
                eid = ebase + i * LANES + lax.iota(jnp.int32, LANES)
                pos = cc + plsc.cumsum(inc) - 1
                plsc.store_scatter(sel_eid, [pos], eid, mask=m)
                plsc.store_scatter(sel_src, [pos], sv, mask=m)
                plsc.store_scatter(sel_dst, [pos], dv, mask=m)

            return cc + nhit

        cnt = lax.fori_loop(0, SEG // LANES, sel_body, cnt)

        nfull = cnt // CH
        process_chunks(nfull)

        rem = cnt - nfull * CH

        @pl.when(nfull > 0)
        def _move():
            for h in range(CH // LANES):
                ev = sel_eid[pl.ds(nfull * CH + h * LANES, LANES)]
                sv = sel_src[pl.ds(nfull * CH + h * LANES, LANES)]
                dv = sel_dst[pl.ds(nfull * CH + h * LANES, LANES)]
                sel_eid[pl.ds(h * LANES, LANES)] = ev
                sel_src[pl.ds(h * LANES, LANES)] = sv
                sel_dst[pl.ds(h * LANES, LANES)] = dv

        return rem

    cnt = lax.fori_loop(0, NSEG, seg_body, jnp.int32(0))

    zpad = jnp.zeros((LANES,), jnp.int32)
    tpad = jnp.full((LANES,), lo + WIN, jnp.int32)
    for h in range(CH // LANES):
        sel_eid[pl.ds(cnt + h * LANES, LANES)] = zpad
        sel_src[pl.ds(cnt + h * LANES, LANES)] = tpad
        sel_dst[pl.ds(cnt + h * LANES, LANES)] = zpad
    process_chunks((cnt + CH - 1) // CH)

    pltpu.sync_copy(acc.at[pl.ds(0, ACC_W)], acc_h.at[pl.ds(w * ACC_W, ACC_W)])


def _make_sc_kernel():
    return functools.partial(
        pl.kernel,
        mesh=plsc.VectorSubcoreMesh(core_axis_name="c", subcore_axis_name="s"),
        out_type=jax.ShapeDtypeStruct((N_PAD * D,), jnp.float32),
        scratch_types=[
            pltpu.VMEM((2, 2, SEG), jnp.int32),
            pltpu.VMEM((SEG + 2 * CH,), jnp.int32),
            pltpu.VMEM((SEG + 2 * CH,), jnp.int32),
            pltpu.VMEM((SEG + 2 * CH,), jnp.int32),
            pltpu.VMEM((CH,), jnp.int32),
            pltpu.VMEM((CH, PQW), jnp.int32),
            pltpu.VMEM((CH, PQW), jnp.int32),
            pltpu.VMEM((CH, TIW), jnp.int32),
            pltpu.VMEM((ACC_W + D,), jnp.float32),
            pltpu.SemaphoreType.DMA,
            pltpu.SemaphoreType.DMA,
        ],
        compiler_params=pltpu.CompilerParams(needs_layout_passes=False),
    )(_sc_body)


def kernel(x, edge_attr, W_tp, W1, b1, W2, b2, W3, b3, W_post, b_post,
           edge_index):
    x_pad = jnp.pad(x, ((0, N_PAD - N), (0, 0)))
    w_sd = jnp.concatenate([W_tp[:D], W_tp[D:2 * D]], axis=1)
    w_e = W_tp[2 * D:]

    p, q = pl.pallas_call(
        _pq_body,
        grid=(N_PAD // 512,),
        in_specs=[
            pl.BlockSpec((512, D), lambda i: (i, 0)),
            pl.BlockSpec((D, 2 * TP_OUT), lambda i: (0, 0)),
        ],
        out_specs=[
            pl.BlockSpec((512, PQW), lambda i: (i, 0)),
            pl.BlockSpec((512, PQW), lambda i: (i, 0)),
        ],
        out_shape=[
            jax.ShapeDtypeStruct((N_PAD, PQW), jnp.int32),
            jax.ShapeDtypeStruct((N_PAD, PQW), jnp.int32),
        ],
    )(x_pad, w_sd)

    eb = 2000
    t_edges = pl.pallas_call(
        _edge_body,
        grid=(E // eb,),
        in_specs=[
            pl.BlockSpec((eb, DE), lambda i: (i, 0)),
            pl.BlockSpec((DE, TP_OUT), lambda i: (0, 0)),
            pl.BlockSpec((DE, 64), lambda i: (0, 0)),
            pl.BlockSpec((1, 64), lambda i: (0, 0)),
            pl.BlockSpec((64, 64), lambda i: (0, 0)),
            pl.BlockSpec((1, 64), lambda i: (0, 0)),
            pl.BlockSpec((64, D), lambda i: (0, 0)),
            pl.BlockSpec((1, D), lambda i: (0, 0)),
        ],
        out_specs=pl.BlockSpec((eb, TIW), lambda i: (i, 0)),
        out_shape=jax.ShapeDtypeStruct((E, TIW), jnp.int32),
    )(edge_attr, w_e, W1, b1.reshape(1, 64), W2, b2.reshape(1, 64),
      W3, b3.reshape(1, D))

    acc = _make_sc_kernel()(edge_index, p, q, t_edges)
    acc = acc.reshape(N_PAD, D)

    out_pad = pl.pallas_call(
        _post_body,
        grid=(N_PAD // 512,),
        in_specs=[
            pl.BlockSpec((512, D), lambda i: (i, 0)),
            pl.BlockSpec((D, D), lambda i: (0, 0)),
            pl.BlockSpec((1, D), lambda i: (0, 0)),
        ],
        out_specs=pl.BlockSpec((512, D), lambda i: (i, 0)),
        out_shape=jax.ShapeDtypeStruct((N_PAD, D), jnp.float32),
    )(acc, W_post, b_post.reshape(1, D))

    return out_pad[:N]


# speedup vs baseline: 1.4856x; 1.0562x over previous
import functools

import jax
import jax.numpy as jnp
import numpy as np
from jax import lax
from jax.experimental import pallas as pl
from jax.experimental.pallas import tpu as pltpu
from jax.experimental.pallas import tpu_sc as plsc

N = 10000
E = 160000
D = 256
DE = 16
TP_OUT = 384
TW = TP_OUT + D
SCAL = 128
NC = 2
NS = 16
NW = NC * NS
LANES = 16

N_PAD = 10240
WIN = N_PAD // NW
SEG = 640
NSEG = E // SEG
CH = 48
ACC_W = WIN * D
PQW = 256
TIW = 384


def _pack_pairs(v):
    r = v.astype(jnp.bfloat16).astype(jnp.float32)
    bits = jax.lax.bitcast_convert_type(r, jnp.uint32)
    cols = []
    for rg in range(v.shape[1] // 128):
        a = bits[:, 128 * rg:128 * rg + 64]
        b = bits[:, 128 * rg + 64:128 * rg + 128]
        cols.append(jax.lax.shift_right_logical(a, jnp.uint32(16))
                    | (b & jnp.uint32(0xFFFF0000)))
    packed = jnp.concatenate(cols, axis=1)
    return jax.lax.bitcast_convert_type(packed, jnp.int32)


def _pq_body(x_ref, w_ref, p_ref, q_ref):
    v = jnp.dot(x_ref[...], w_ref[...], preferred_element_type=jnp.float32)
    nb = v.shape[0]
    zpad = jnp.zeros((nb, PQW - TP_OUT // 2), jnp.int32)
    p_ref[...] = jnp.concatenate([_pack_pairs(v[:, :TP_OUT]), zpad], axis=1)
    q_ref[...] = jnp.concatenate([_pack_pairs(v[:, TP_OUT:]), zpad], axis=1)


def _edge_body(ea_ref, we_ref, w1_ref, b1_ref, w2_ref, b2_ref, w3_ref,
               b3_ref, t_ref):
    ea = ea_ref[...]
    rr = jnp.dot(ea, we_ref[...], preferred_element_type=jnp.float32)
    h = jax.nn.silu(jnp.dot(ea, w1_ref[...],
                            preferred_element_type=jnp.float32) + b1_ref[...])
    h = jax.nn.silu(jnp.dot(h, w2_ref[...],
                            preferred_element_type=jnp.float32) + b2_ref[...])
    ww = (jnp.dot(h, w3_ref[...], preferred_element_type=jnp.float32)
          + b3_ref[...])
    nb = ea.shape[0]
    zpad = jnp.zeros((nb, TIW - TW // 2), jnp.int32)
    t_ref[...] = jnp.concatenate(
        [_pack_pairs(rr), _pack_pairs(ww), zpad], axis=1)


def _post_body(a_ref, w_ref, b_ref, o_ref):
    o_ref[...] = jnp.dot(a_ref[...], w_ref[...],
                         preferred_element_type=jnp.float32) + b_ref[...]


def _sc_body(ei_h, p_h, q_h, t_h, acc_h,
             eiv, sel_eid, sel_src, sel_dst, gidx_v,
             buf_p, buf_q, buf_t, acc, sem, sem2):
    c = lax.axis_index("c")
    s = lax.axis_index("s")
    w = c * NS + s
    lo = w * WIN

    pltpu.async_copy(ei_h.at[:, pl.ds(0, SEG)], eiv.at[0], sem2)

    zvec = jnp.zeros((LANES,), jnp.float32)

    def zero_body(i, _):
        acc[pl.ds(i * LANES, LANES)] = zvec
        return 0

    lax.fori_loop(0, (ACC_W + D) // LANES, zero_body, 0)

    def process_chunks(nch):
        def chunk_body(ci, _):
            for h in range(CH // LANES):
                sv = sel_src[pl.ds(ci * CH + h * LANES, LANES)]
                gidx_v[pl.ds(h * LANES, LANES)] = jnp.minimum(
                    sv, jnp.int32(N_PAD - 1))
            cp_p = pltpu.async_copy(p_h.at[gidx_v], buf_p, sem)
            cp_q = pltpu.async_copy(
                q_h.at[sel_dst.at[pl.ds(ci * CH, CH)]], buf_q, sem)
            cp_t = pltpu.async_copy(
                t_h.at[sel_eid.at[pl.ds(ci * CH, CH)]], buf_t, sem)
            cp_p.wait()
            cp_q.wait()
            cp_t.wait()

            def unpk(ref, j, col):
                v = plsc.bitcast(ref[j, pl.ds(col, LANES)], jnp.bfloat16)
                return plsc.unpack(v, format=plsc.PackFormat.INTERLEAVED,
                                   preferred_element_type=jnp.float32)

            def edge_body(j, _):
                rv = sel_src[pl.ds(ci * CH + j, LANES)]
                base = (rv[0] - lo) * D
                for g in range(4):
                    p0, p1 = unpk(buf_p, j, 16 * g)
                    q0, q1 = unpk(buf_q, j, 16 * g)
                    t0, t1 = unpk(buf_t, j, 16 * g)
                    w0, w1 = unpk(buf_t, j, TP_OUT // 2 + 16 * g)
                    u0 = p0 + q0 + t0
                    u1 = p1 + q1 + t1
                    y0 = u0 / (1.0 + jnp.exp(-u0)) * w0
                    y1 = u1 / (1.0 + jnp.exp(-u1)) * w1
                    plsc.addupdate(acc.at[pl.ds(base + 16 * g, LANES)], y0)
                    plsc.addupdate(
                        acc.at[pl.ds(base + 16 * g + 64, LANES)], y1)
                for g in range(4):
                    pg0, pg1 = unpk(buf_p, j, SCAL // 2 + 16 * g)
                    qg0, qg1 = unpk(buf_q, j, SCAL // 2 + 16 * g)
                    tg0, tg1 = unpk(buf_t, j, SCAL // 2 + 16 * g)
                    pd0, pd1 = unpk(buf_p, j, SCAL + 16 * g)
                    qd0, qd1 = unpk(buf_q, j, SCAL + 16 * g)
                    td0, td1 = unpk(buf_t, j, SCAL + 16 * g)
                    w0, w1 = unpk(buf_t, j, (TP_OUT + SCAL) // 2 + 16 * g)
                    ug0 = pg0 + qg0 + tg0
                    ug1 = pg1 + qg1 + tg1
                    ud0 = pd0 + qd0 + td0
                    ud1 = pd1 + qd1 + td1
                    y0 = ud0 * w0 / (1.0 + jnp.exp(-ug0))
                    y1 = ud1 * w1 / (1.0 + jnp.exp(-ug1))
                    plsc.addupdate(
                        acc.at[pl.ds(base + SCAL + 16 * g, LANES)], y0)
                    plsc.addupdate(
                        acc.at[pl.ds(base + SCAL + 16 * g + 64, LANES)], y1)
                return 0

            lax.fori_loop(0, CH, edge_body, 0)
            return 0

        lax.fori_loop(0, nch, chunk_body, 0)

    def seg_body(g, cnt):
        ebase = g * SEG
        b = lax.rem(g, 2)
        pltpu.make_async_copy(ei_h.at[:, pl.ds(ebase, SEG)],
                              eiv.at[b], sem2).wait()

        @pl.when(g + 1 < NSEG)
        def _prefetch():
            pltpu.async_copy(ei_h.at[:, pl.ds(ebase + SEG, SEG)],
                             eiv.at[lax.rem(g + 1, 2)], sem2)

        def sel_body(i, cc):
            sv1 = eiv[b, 0, pl.ds(2 * i * LANES, LANES)]
            sv2 = eiv[b, 0, pl.ds((2 * i + 1) * LANES, LANES)]
            m1 = (sv1 >= lo) & (sv1 < lo + WIN)
            m2 = (sv2 >= lo) & (sv2 < lo + WIN)
            inc1 = m1.astype(jnp.int32)
            inc2 = m2.astype(jnp.int32)
            n1 = jnp.sum(inc1)
            n2 = jnp.sum(inc2)

            @pl.when(n1 + n2 > 0)
            def _store():
                dv1 = eiv[b, 1, pl.ds(2 * i * LANES, LANES)]
                dv2 = eiv[b, 1, pl.ds((2 * i + 1) * LANES, LANES)]
                eid1 = ebase + 2 * i * LANES + lax.iota(jnp.int32, LANES)
                eid2 = eid1 + LANES
                pos1 = cc + plsc.cumsum(inc1) - 1
                pos2 = cc + n1 + plsc.cumsum(inc2) - 1
                plsc.store_scatter(sel_eid, [pos1], eid1, mask=m1)
                plsc.store_scatter(sel_src, [pos1], sv1, mask=m1)
                plsc.store_scatter(sel_dst, [pos1], dv1, mask=m1)
                plsc.store_scatter(sel_eid, [pos2], eid2, mask=m2)
                plsc.store_scatter(sel_src, [pos2], sv2, mask=m2)
                plsc.store_scatter(sel_dst, [pos2], dv2, mask=m2)

            return cc + n1 + n2

        cnt = lax.fori_loop(0, SEG // (2 * LANES), sel_body, cnt)

        nfull = cnt // CH
        process_chunks(nfull)

        rem = cnt - nfull * CH

        @pl.when(nfull > 0)
        def _move():
            for h in range(CH // LANES):
                ev = sel_eid[pl.ds(nfull * CH + h * LANES, LANES)]
                sv = sel_src[pl.ds(nfull * CH + h * LANES, LANES)]
                dv = sel_dst[pl.ds(nfull * CH + h * LANES, LANES)]
                sel_eid[pl.ds(h * LANES, LANES)] = ev
                sel_src[pl.ds(h * LANES, LANES)] = sv
                sel_dst[pl.ds(h * LANES, LANES)] = dv

        return rem

    cnt = lax.fori_loop(0, NSEG, seg_body, jnp.int32(0))

    zpad = jnp.zeros((LANES,), jnp.int32)
    tpad = jnp.full((LANES,), lo + WIN, jnp.int32)
    for h in range(CH // LANES):
        sel_eid[pl.ds(cnt + h * LANES, LANES)] = zpad
        sel_src[pl.ds(cnt + h * LANES, LANES)] = tpad
        sel_dst[pl.ds(cnt + h * LANES, LANES)] = zpad
    process_chunks((cnt + CH - 1) // CH)

    pltpu.sync_copy(acc.at[pl.ds(0, ACC_W)], acc_h.at[pl.ds(w * ACC_W, ACC_W)])


def _make_sc_kernel():
    return functools.partial(
        pl.kernel,
        mesh=plsc.VectorSubcoreMesh(core_axis_name="c", subcore_axis_name="s"),
        out_type=jax.ShapeDtypeStruct((N_PAD * D,), jnp.float32),
        scratch_types=[
            pltpu.VMEM((2, 2, SEG), jnp.int32),
            pltpu.VMEM((SEG + 2 * CH,), jnp.int32),
            pltpu.VMEM((SEG + 2 * CH,), jnp.int32),
            pltpu.VMEM((SEG + 2 * CH,), jnp.int32),
            pltpu.VMEM((CH,), jnp.int32),
            pltpu.VMEM((CH, PQW), jnp.int32),
            pltpu.VMEM((CH, PQW), jnp.int32),
            pltpu.VMEM((CH, TIW), jnp.int32),
            pltpu.VMEM((ACC_W + D,), jnp.float32),
            pltpu.SemaphoreType.DMA,
            pltpu.SemaphoreType.DMA,
        ],
        compiler_params=pltpu.CompilerParams(needs_layout_passes=False),
    )(_sc_body)


def kernel(x, edge_attr, W_tp, W1, b1, W2, b2, W3, b3, W_post, b_post,
           edge_index):
    x_pad = jnp.pad(x, ((0, N_PAD - N), (0, 0)))
    w_sd = jnp.concatenate([W_tp[:D], W_tp[D:2 * D]], axis=1)
    w_e = W_tp[2 * D:]

    p, q = pl.pallas_call(
        _pq_body,
        grid=(N_PAD // 512,),
        in_specs=[
            pl.BlockSpec((512, D), lambda i: (i, 0)),
            pl.BlockSpec((D, 2 * TP_OUT), lambda i: (0, 0)),
        ],
        out_specs=[
            pl.BlockSpec((512, PQW), lambda i: (i, 0)),
            pl.BlockSpec((512, PQW), lambda i: (i, 0)),
        ],
        out_shape=[
            jax.ShapeDtypeStruct((N_PAD, PQW), jnp.int32),
            jax.ShapeDtypeStruct((N_PAD, PQW), jnp.int32),
        ],
    )(x_pad, w_sd)

    eb = 2000
    t_edges = pl.pallas_call(
        _edge_body,
        grid=(E // eb,),
        in_specs=[
            pl.BlockSpec((eb, DE), lambda i: (i, 0)),
            pl.BlockSpec((DE, TP_OUT), lambda i: (0, 0)),
            pl.BlockSpec((DE, 64), lambda i: (0, 0)),
            pl.BlockSpec((1, 64), lambda i: (0, 0)),
            pl.BlockSpec((64, 64), lambda i: (0, 0)),
            pl.BlockSpec((1, 64), lambda i: (0, 0)),
            pl.BlockSpec((64, D), lambda i: (0, 0)),
            pl.BlockSpec((1, D), lambda i: (0, 0)),
        ],
        out_specs=pl.BlockSpec((eb, TIW), lambda i: (i, 0)),
        out_shape=jax.ShapeDtypeStruct((E, TIW), jnp.int32),
    )(edge_attr, w_e, W1, b1.reshape(1, 64), W2, b2.reshape(1, 64),
      W3, b3.reshape(1, D))

    acc = _make_sc_kernel()(edge_index, p, q, t_edges)
    acc = acc.reshape(N_PAD, D)

    out_pad = pl.pallas_call(
        _post_body,
        grid=(N_PAD // 512,),
        in_specs=[
            pl.BlockSpec((512, D), lambda i: (i, 0)),
            pl.BlockSpec((D, D), lambda i: (0, 0)),
            pl.BlockSpec((1, D), lambda i: (0, 0)),
        ],
        out_specs=pl.BlockSpec((512, D), lambda i: (i, 0)),
        out_shape=jax.ShapeDtypeStruct((N_PAD, D), jnp.float32),
    )(acc, W_post, b_post.reshape(1, D))

    return out_pad[:N]


# speedup vs baseline: 1.5094x; 1.0160x over previous
import functools

import jax
import jax.numpy as jnp
import numpy as np
from jax import lax
from jax.experimental import pallas as pl
from jax.experimental.pallas import tpu as pltpu
from jax.experimental.pallas import tpu_sc as plsc

N = 10000
E = 160000
D = 256
DE = 16
TP_OUT = 384
TW = TP_OUT + D
SCAL = 128
NC = 2
NS = 16
NW = NC * NS
LANES = 16

N_PAD = 10240
WIN = N_PAD // NW
SEG = 640
NSEG = E // SEG
CH = 48
ACC_W = WIN * D
PQW = 256
TIW = 384


def _pack_pairs(v):
    r = v.astype(jnp.bfloat16).astype(jnp.float32)
    bits = jax.lax.bitcast_convert_type(r, jnp.uint32)
    cols = []
    for rg in range(v.shape[1] // 128):
        a = bits[:, 128 * rg:128 * rg + 64]
        b = bits[:, 128 * rg + 64:128 * rg + 128]
        cols.append(jax.lax.shift_right_logical(a, jnp.uint32(16))
                    | (b & jnp.uint32(0xFFFF0000)))
    packed = jnp.concatenate(cols, axis=1)
    return jax.lax.bitcast_convert_type(packed, jnp.int32)


def _pq_body(x_ref, w_ref, p_ref, q_ref):
    v = jnp.dot(x_ref[...], w_ref[...], preferred_element_type=jnp.float32)
    nb = v.shape[0]
    zpad = jnp.zeros((nb, PQW - TP_OUT // 2), jnp.int32)
    p_ref[...] = jnp.concatenate([_pack_pairs(v[:, :TP_OUT]), zpad], axis=1)
    q_ref[...] = jnp.concatenate([_pack_pairs(v[:, TP_OUT:]), zpad], axis=1)


def _edge_body(ea_ref, we_ref, w1_ref, b1_ref, w2_ref, b2_ref, w3_ref,
               b3_ref, t_ref):
    ea = ea_ref[...]
    rr = jnp.dot(ea, we_ref[...], preferred_element_type=jnp.float32)
    h = jax.nn.silu(jnp.dot(ea, w1_ref[...],
                            preferred_element_type=jnp.float32) + b1_ref[...])
    h = jax.nn.silu(jnp.dot(h, w2_ref[...],
                            preferred_element_type=jnp.float32) + b2_ref[...])
    ww = (jnp.dot(h, w3_ref[...], preferred_element_type=jnp.float32)
          + b3_ref[...])
    nb = ea.shape[0]
    zpad = jnp.zeros((nb, TIW - TW // 2), jnp.int32)
    t_ref[...] = jnp.concatenate(
        [_pack_pairs(rr), _pack_pairs(ww), zpad], axis=1)


def _post_body(a_ref, w_ref, b_ref, o_ref):
    o_ref[...] = jnp.dot(a_ref[...], w_ref[...],
                         preferred_element_type=jnp.float32) + b_ref[...]


def _sc_body(ei_h, p_h, q_h, t_h, acc_h,
             eiv, sel_eid, sel_src, sel_dst, gidx_v,
             buf_p, buf_q, buf_t, acc, sem, sem2):
    c = lax.axis_index("c")
    s = lax.axis_index("s")
    w = c * NS + s
    lo = w * WIN

    pltpu.async_copy(ei_h.at[:, pl.ds(0, SEG)], eiv.at[0], sem2)

    zvec = jnp.zeros((LANES,), jnp.float32)

    def zero_body(i, _):
        acc[pl.ds(i * LANES, LANES)] = zvec
        return 0

    lax.fori_loop(0, (ACC_W + D) // LANES, zero_body, 0)

    def process_chunks(nch):
        def chunk_body(ci, _):
            for h in range(CH // LANES):
                sv = sel_src[pl.ds(ci * CH + h * LANES, LANES)]
                gidx_v[pl.ds(h * LANES, LANES)] = jnp.minimum(
                    sv, jnp.int32(N_PAD - 1))
            cp_p = pltpu.async_copy(p_h.at[gidx_v], buf_p, sem)
            cp_q = pltpu.async_copy(
                q_h.at[sel_dst.at[pl.ds(ci * CH, CH)]], buf_q, sem)
            cp_t = pltpu.async_copy(
                t_h.at[sel_eid.at[pl.ds(ci * CH, CH)]], buf_t, sem)
            cp_p.wait()
            cp_q.wait()
            cp_t.wait()

            def unpk(ref, j, col):
                v = plsc.bitcast(ref[j, pl.ds(col, LANES)], jnp.bfloat16)
                return plsc.unpack(v, format=plsc.PackFormat.INTERLEAVED,
                                   preferred_element_type=jnp.float32)

            def edge_body(j, _):
                rv = sel_src[pl.ds(ci * CH + j, LANES)]
                base = (rv[0] - lo) * D
                for g in range(4):
                    p0, p1 = unpk(buf_p, j, 16 * g)
                    q0, q1 = unpk(buf_q, j, 16 * g)
                    t0, t1 = unpk(buf_t, j, 16 * g)
                    w0, w1 = unpk(buf_t, j, TP_OUT // 2 + 16 * g)
                    u0 = p0 + q0 + t0
                    u1 = p1 + q1 + t1
                    y0 = u0 / (1.0 + jnp.exp(-u0)) * w0
                    y1 = u1 / (1.0 + jnp.exp(-u1)) * w1
                    plsc.addupdate(acc.at[pl.ds(base + 16 * g, LANES)], y0)
                    plsc.addupdate(
                        acc.at[pl.ds(base + 16 * g + 64, LANES)], y1)
                for g in range(4):
                    pg0, pg1 = unpk(buf_p, j, SCAL // 2 + 16 * g)
                    qg0, qg1 = unpk(buf_q, j, SCAL // 2 + 16 * g)
                    tg0, tg1 = unpk(buf_t, j, SCAL // 2 + 16 * g)
                    pd0, pd1 = unpk(buf_p, j, SCAL + 16 * g)
                    qd0, qd1 = unpk(buf_q, j, SCAL + 16 * g)
                    td0, td1 = unpk(buf_t, j, SCAL + 16 * g)
                    w0, w1 = unpk(buf_t, j, (TP_OUT + SCAL) // 2 + 16 * g)
                    ug0 = pg0 + qg0 + tg0
                    ug1 = pg1 + qg1 + tg1
                    ud0 = pd0 + qd0 + td0
                    ud1 = pd1 + qd1 + td1
                    y0 = ud0 * w0 / (1.0 + jnp.exp(-ug0))
                    y1 = ud1 * w1 / (1.0 + jnp.exp(-ug1))
                    plsc.addupdate(
                        acc.at[pl.ds(base + SCAL + 16 * g, LANES)], y0)
                    plsc.addupdate(
                        acc.at[pl.ds(base + SCAL + 16 * g + 64, LANES)], y1)
                return 0

            lax.fori_loop(0, CH, edge_body, 0)
            return 0

        lax.fori_loop(0, nch, chunk_body, 0)

    def seg_body(g, cnt):
        ebase = g * SEG
        b = lax.rem(g, 2)
        pltpu.make_async_copy(ei_h.at[:, pl.ds(ebase, SEG)],
                              eiv.at[b], sem2).wait()

        @pl.when(g + 1 < NSEG)
        def _prefetch():
            pltpu.async_copy(ei_h.at[:, pl.ds(ebase + SEG, SEG)],
                             eiv.at[lax.rem(g + 1, 2)], sem2)

        NV = 4

        def sel_body(i, cc):
            svs = [eiv[b, 0, pl.ds((NV * i + k) * LANES, LANES)]
                   for k in range(NV)]
            ms = [(sv >= lo) & (sv < lo + WIN) for sv in svs]
            incs = [m.astype(jnp.int32) for m in ms]
            ns = [jnp.sum(inc) for inc in incs]
            tot = ns[0] + ns[1] + ns[2] + ns[3]

            @pl.when(tot > 0)
            def _store():
                off = cc
                for k in range(NV):
                    dv = eiv[b, 1, pl.ds((NV * i + k) * LANES, LANES)]
                    eid = (ebase + (NV * i + k) * LANES
                           + lax.iota(jnp.int32, LANES))
                    pos = off + plsc.cumsum(incs[k]) - 1
                    plsc.store_scatter(sel_eid, [pos], eid, mask=ms[k])
                    plsc.store_scatter(sel_src, [pos], svs[k], mask=ms[k])
                    plsc.store_scatter(sel_dst, [pos], dv, mask=ms[k])
                    off = off + ns[k]

            return cc + tot

        cnt = lax.fori_loop(0, SEG // (NV * LANES), sel_body, cnt)

        nfull = cnt // CH
        process_chunks(nfull)

        rem = cnt - nfull * CH

        @pl.when(nfull > 0)
        def _move():
            for h in range(CH // LANES):
                ev = sel_eid[pl.ds(nfull * CH + h * LANES, LANES)]
                sv = sel_src[pl.ds(nfull * CH + h * LANES, LANES)]
                dv = sel_dst[pl.ds(nfull * CH + h * LANES, LANES)]
                sel_eid[pl.ds(h * LANES, LANES)] = ev
                sel_src[pl.ds(h * LANES, LANES)] = sv
                sel_dst[pl.ds(h * LANES, LANES)] = dv

        return rem

    cnt = lax.fori_loop(0, NSEG, seg_body, jnp.int32(0))

    zpad = jnp.zeros((LANES,), jnp.int32)
    tpad = jnp.full((LANES,), lo + WIN, jnp.int32)
    for h in range(CH // LANES):
        sel_eid[pl.ds(cnt + h * LANES, LANES)] = zpad
        sel_src[pl.ds(cnt + h * LANES, LANES)] = tpad
        sel_dst[pl.ds(cnt + h * LANES, LANES)] = zpad
    process_chunks((cnt + CH - 1) // CH)

    pltpu.sync_copy(acc.at[pl.ds(0, ACC_W)], acc_h.at[pl.ds(w * ACC_W, ACC_W)])


def _make_sc_kernel():
    return functools.partial(
        pl.kernel,
        mesh=plsc.VectorSubcoreMesh(core_axis_name="c", subcore_axis_name="s"),
        out_type=jax.ShapeDtypeStruct((N_PAD * D,), jnp.float32),
        scratch_types=[
            pltpu.VMEM((2, 2, SEG), jnp.int32),
            pltpu.VMEM((SEG + 2 * CH,), jnp.int32),
            pltpu.VMEM((SEG + 2 * CH,), jnp.int32),
            pltpu.VMEM((SEG + 2 * CH,), jnp.int32),
            pltpu.VMEM((CH,), jnp.int32),
            pltpu.VMEM((CH, PQW), jnp.int32),
            pltpu.VMEM((CH, PQW), jnp.int32),
            pltpu.VMEM((CH, TIW), jnp.int32),
            pltpu.VMEM((ACC_W + D,), jnp.float32),
            pltpu.SemaphoreType.DMA,
            pltpu.SemaphoreType.DMA,
        ],
        compiler_params=pltpu.CompilerParams(needs_layout_passes=False),
    )(_sc_body)


def kernel(x, edge_attr, W_tp, W1, b1, W2, b2, W3, b3, W_post, b_post,
           edge_index):
    x_pad = jnp.pad(x, ((0, N_PAD - N), (0, 0)))
    w_sd = jnp.concatenate([W_tp[:D], W_tp[D:2 * D]], axis=1)
    w_e = W_tp[2 * D:]

    p, q = pl.pallas_call(
        _pq_body,
        grid=(N_PAD // 512,),
        in_specs=[
            pl.BlockSpec((512, D), lambda i: (i, 0)),
            pl.BlockSpec((D, 2 * TP_OUT), lambda i: (0, 0)),
        ],
        out_specs=[
            pl.BlockSpec((512, PQW), lambda i: (i, 0)),
            pl.BlockSpec((512, PQW), lambda i: (i, 0)),
        ],
        out_shape=[
            jax.ShapeDtypeStruct((N_PAD, PQW), jnp.int32),
            jax.ShapeDtypeStruct((N_PAD, PQW), jnp.int32),
        ],
    )(x_pad, w_sd)

    eb = 2000
    t_edges = pl.pallas_call(
        _edge_body,
        grid=(E // eb,),
        in_specs=[
            pl.BlockSpec((eb, DE), lambda i: (i, 0)),
            pl.BlockSpec((DE, TP_OUT), lambda i: (0, 0)),
            pl.BlockSpec((DE, 64), lambda i: (0, 0)),
            pl.BlockSpec((1, 64), lambda i: (0, 0)),
            pl.BlockSpec((64, 64), lambda i: (0, 0)),
            pl.BlockSpec((1, 64), lambda i: (0, 0)),
            pl.BlockSpec((64, D), lambda i: (0, 0)),
            pl.BlockSpec((1, D), lambda i: (0, 0)),
        ],
        out_specs=pl.BlockSpec((eb, TIW), lambda i: (i, 0)),
        out_shape=jax.ShapeDtypeStruct((E, TIW), jnp.int32),
    )(edge_attr, w_e, W1, b1.reshape(1, 64), W2, b2.reshape(1, 64),
      W3, b3.reshape(1, D))

    acc = _make_sc_kernel()(edge_index, p, q, t_edges)
    acc = acc.reshape(N_PAD, D)

    out_pad = pl.pallas_call(
        _post_body,
        grid=(N_PAD // 512,),
        in_specs=[
            pl.BlockSpec((512, D), lambda i: (i, 0)),
            pl.BlockSpec((D, D), lambda i: (0, 0)),
            pl.BlockSpec((1, D), lambda i: (0, 0)),
        ],
        out_specs=pl.BlockSpec((512, D), lambda i: (i, 0)),
        out_shape=jax.ShapeDtypeStruct((N_PAD, D), jnp.float32),
    )(acc, W_post, b_post.reshape(1, D))

    return out_pad[:N]
